# Initial kernel scaffold; baseline (speedup 1.0000x reference)
import jax
import jax.numpy as jnp
from jax.experimental import pallas as pl


def kernel(x, edge_index, edge_attr, W1, We1, as1, ad1, ae1, b1, W2, We2, as2, ad2, ae2, b2, W3, We3, as3, ad3, ae3, b3):
    raise NotImplementedError("write your pallas kernel here")


# speedup vs baseline: 1.0583x; 1.0583x over previous
import functools

import jax
import jax.numpy as jnp
from jax import lax
from jax.experimental import pallas as pl
from jax.experimental.pallas import tpu as pltpu

N = 10000
E = 320000
HEADS = 8
C = 128
HC = HEADS * C
EDGE_DIM = 32

_M_TILE = 400
_E_TILE = 2560


def _fold_weights(W, We, a_s, a_d, a_e):
    d = W.shape[0]
    Was = (W.reshape(d, HEADS, C) * a_s[None]).sum(-1)
    Wad = (W.reshape(d, HEADS, C) * a_d[None]).sum(-1)
    Ve = (We.reshape(EDGE_DIM, HEADS, C) * a_e[None]).sum(-1)
    pad = jnp.zeros((d, 1152 - HC - 16), jnp.float32)
    Wcat = jnp.concatenate([W, Was, Wad, pad], axis=1)
    return Wcat, Ve


def _mm_body(x_ref, w_ref, o_ref, *, relu):
    xb = x_ref[...]
    if relu:
        xb = jnp.maximum(xb, 0.0)
    o_ref[...] = jnp.dot(xb, w_ref[...], preferred_element_type=jnp.float32)


def _node_matmul(xin, Wcat, relu):
    K = xin.shape[1]
    return pl.pallas_call(
        functools.partial(_mm_body, relu=relu),
        grid=(N // _M_TILE,),
        in_specs=[
            pl.BlockSpec((_M_TILE, K), lambda i: (i, 0)),
            pl.BlockSpec((K, 1152), lambda i: (0, 0)),
        ],
        out_specs=pl.BlockSpec((_M_TILE, 1152), lambda i: (i, 0)),
        out_shape=jax.ShapeDtypeStruct((N, 1152), jnp.float32),
    )(xin, Wcat)


def _ae_body(v_ref, ea_ref, o_ref):
    o_ref[...] = lax.dot_general(
        v_ref[...], ea_ref[...], (((0,), (1,)), ((), ())),
        preferred_element_type=jnp.float32)


def _edge_logits(Vcat, edge_attr):
    return pl.pallas_call(
        _ae_body,
        grid=(E // _E_TILE,),
        in_specs=[
            pl.BlockSpec((EDGE_DIM, 24), lambda i: (0, 0)),
            pl.BlockSpec((_E_TILE, EDGE_DIM), lambda i: (i, 0)),
        ],
        out_specs=pl.BlockSpec((24, _E_TILE), lambda i: (0, i)),
        out_shape=jax.ShapeDtypeStruct((24, E), jnp.float32),
    )(Vcat, edge_attr)


def _gat_layer_jax(h, asrc, adst, aet, src, dst, bias, concat):
    alpha = asrc[src] + adst[dst] + aet.T
    alpha = jnp.where(alpha > 0, alpha, 0.2 * alpha)
    ex = jnp.exp(alpha)
    denom = jax.ops.segment_sum(ex, dst, num_segments=N)
    msg = h[src].reshape(E, HEADS, C) * ex[:, :, None]
    acc = jax.ops.segment_sum(msg, dst, num_segments=N)
    out = acc / (denom[:, :, None] + 1e-16)
    if concat:
        out = out.reshape(N, HC)
    else:
        out = out.mean(axis=1)
    return out + bias


def kernel(x, edge_index, edge_attr,
           W1, We1, as1, ad1, ae1, b1,
           W2, We2, as2, ad2, ae2, b2,
           W3, We3, as3, ad3, ae3, b3):
    src = edge_index[0]
    dst = edge_index[1]

    Wc1, Ve1 = _fold_weights(W1, We1, as1, ad1, ae1)
    Wc2, Ve2 = _fold_weights(W2, We2, as2, ad2, ae2)
    Wc3, Ve3 = _fold_weights(W3, We3, as3, ad3, ae3)
    Vcat = jnp.concatenate([Ve1, Ve2, Ve3], axis=1)

    aet3 = _edge_logits(Vcat, edge_attr)

    xin = x
    for li, (Wc, bias) in enumerate(((Wc1, b1), (Wc2, b2), (Wc3, b3))):
        h_all = _node_matmul(xin, Wc, relu=(li > 0))
        h = h_all[:, :HC]
        asrc = h_all[:, HC:HC + 8]
        adst = h_all[:, HC + 8:HC + 16]
        aet = lax.dynamic_slice_in_dim(aet3, li * 8, 8, axis=0)
        xin = _gat_layer_jax(h, asrc, adst, aet, src, dst, bias,
                             concat=(li < 2))
    return xin


# speedup vs baseline: 5.5095x; 5.2061x over previous
import functools

import jax
import jax.numpy as jnp
from jax import lax
from jax.experimental import pallas as pl
from jax.experimental.pallas import tpu as pltpu
from jax.experimental.pallas import tpu_sc as plsc

N = 10000
E = 320000
HEADS = 8
C = 128
HC = HEADS * C
EDGE_DIM = 32

_M_TILE = 400
_E_TILE = 2048

_NC = 2
_NS = 16
_NW = _NC * _NS

_BS = 256
_NB = 40
_EP = 393216
_EPT = _EP // _NW
_K3 = 1024
_NCH3 = _EPT // _K3
_K4 = 64
_NP = 10240
_DUMMY = N

_sc_mesh = plsc.VectorSubcoreMesh(
    core_axis_name="c", subcore_axis_name="s", num_cores=_NC, num_subcores=_NS)
_sc_params = pltpu.CompilerParams(use_tc_tiling_on_sc=False)


def _i16():
    return lax.broadcasted_iota(jnp.int32, (16,), 0)


def _spl(v):
    return jnp.full((16,), v, jnp.int32)


def _comb8(a, b):
    i = _i16()
    return jnp.where(i < 8, a, jnp.take(b, i & 7))


def _fold_weights(W, We, a_s, a_d, a_e):
    d = W.shape[0]
    Was = (W.reshape(d, HEADS, C) * a_s[None]).sum(-1)
    Wad = (W.reshape(d, HEADS, C) * a_d[None]).sum(-1)
    Ve = (We.reshape(EDGE_DIM, HEADS, C) * a_e[None]).sum(-1)
    z8 = jnp.zeros((d, 8), jnp.float32)
    pad = jnp.zeros((d, 1152 - HC - 32), jnp.float32)
    Wcat = jnp.concatenate([W, Was, z8, Wad, z8, pad], axis=1)
    Ve16 = jnp.concatenate([Ve, jnp.zeros((EDGE_DIM, 8), jnp.float32)], axis=1)
    return Wcat, Ve16


def _mm_body(x_ref, w_ref, o_ref, *, relu):
    xb = x_ref[...]
    if relu:
        xb = jnp.maximum(xb, 0.0)
    o_ref[...] = jnp.dot(xb, w_ref[...], preferred_element_type=jnp.float32)


def _node_matmul(xin, Wcat, relu):
    K = xin.shape[1]
    return pl.pallas_call(
        functools.partial(_mm_body, relu=relu),
        grid=(N // _M_TILE,),
        in_specs=[
            pl.BlockSpec((_M_TILE, K), lambda i: (i, 0)),
            pl.BlockSpec((K, 1152), lambda i: (0, 0)),
        ],
        out_specs=pl.BlockSpec((_M_TILE, 1152), lambda i: (i, 0)),
        out_shape=jax.ShapeDtypeStruct((N, 1152), jnp.float32),
    )(xin, Wcat)


def _ae_body(ea_ref, v_ref, o_ref):
    o_ref[...] = jnp.dot(ea_ref[...], v_ref[...],
                         preferred_element_type=jnp.float32)


def _edge_logits(eap, Vcat):
    return pl.pallas_call(
        _ae_body,
        grid=(_EP // _E_TILE,),
        in_specs=[
            pl.BlockSpec((_E_TILE, EDGE_DIM), lambda i: (i, 0)),
            pl.BlockSpec((EDGE_DIM, 48), lambda i: (0, 0)),
        ],
        out_specs=pl.BlockSpec((_E_TILE, 48), lambda i: (i, 0)),
        out_shape=jax.ShapeDtypeStruct((_EP, 48), jnp.float32),
    )(eap, Vcat)


def _alpha_body(srcp, dstp, aet, asrc, adst,
                exp_out, den_out,
                src_c, dst_c, ae_c, asg, adg, ex_s, didx_c, zrow, den_sp):
    c = lax.axis_index("c")
    s = lax.axis_index("s")
    zero16 = jnp.zeros((16,), jnp.float32)

    def _z(i, _):
        zrow[pl.ds(i * 16, 16)] = zero16
        return 0
    lax.fori_loop(0, 320, _z, 0)
    pltpu.sync_copy(zrow, den_sp.at[pl.ds(s * 5120, 5120)])
    plsc.subcore_barrier()

    gbase = c * (_EP // _NC) + s * _EPT
    i16 = _i16()
    lane7 = i16 & 7

    def _chunk(i, _):
        b = gbase + i * _K3
        br = b // 128
        pltpu.sync_copy(srcp.at[pl.ds(br, 8)], src_c)
        pltpu.sync_copy(dstp.at[pl.ds(br, 8)], dst_c)
        pltpu.sync_copy(aet.at[pl.ds(b, _K3)], ae_c)
        for t in range(8):
            pltpu.sync_copy(asrc.at[src_c.at[t]], asg.at[pl.ds(t * 128, 128)])
        for t in range(8):
            pltpu.sync_copy(adst.at[dst_c.at[t]], adg.at[pl.ds(t * 128, 128)])

        def _pair(m, _):
            av = _comb8(asg[2 * m], asg[2 * m + 1])
            bv = _comb8(adg[2 * m], adg[2 * m + 1])
            ev = _comb8(ae_c[2 * m], ae_c[2 * m + 1])
            al = av + bv + ev
            al = jnp.where(al > 0, al, al * 0.2)
            exv = jnp.exp(al)
            ex_s[pl.ds(m * 16, 16)] = exv
            dstv = dst_c[m >> 6, pl.ds(((m >> 3) & 7) * 16, 16)]
            e0 = (m & 7) * 2
            d0 = jnp.take(dstv, _spl(0) + e0)
            d1 = jnp.take(dstv, _spl(0) + (e0 + 1))
            didx = jnp.where(i16 < 8, d0 * 8 + i16, d1 * 8 + lane7)
            didx_c[m >> 3, pl.ds((m & 7) * 16, 16)] = didx
            return 0
        lax.fori_loop(0, _K3 // 2, _pair, 0)

        def _scat(t, _):
            pltpu.sync_copy(ex_s.at[pl.ds(t * 128, 128)],
                            den_sp.at[didx_c.at[t]], add=True)
            return 0
        lax.fori_loop(0, _K3 * 8 // 128, _scat, 0)
        pltpu.sync_copy(ex_s, exp_out.at[pl.ds(b * 8, _K3 * 8)])
        return 0

    lax.fori_loop(0, _NCH3, _chunk, 0)
    plsc.subcore_barrier()

    @pl.when(s == 0)
    def _():
        pltpu.sync_copy(den_sp, den_out.at[pl.ds(c * 8 * _NP, 8 * _NP)])


def _alpha_pass(srcp, dstp, aet, asrc, adst):
    f = pl.kernel(
        _alpha_body,
        out_type=(jax.ShapeDtypeStruct((_EP * 8,), jnp.float32),
                  jax.ShapeDtypeStruct((2 * 8 * _NP,), jnp.float32)),
        mesh=_sc_mesh,
        compiler_params=_sc_params,
        scratch_types=[
            pltpu.VMEM((8, 128), jnp.int32),
            pltpu.VMEM((8, 128), jnp.int32),
            pltpu.VMEM((_K3, 16), jnp.float32),
            pltpu.VMEM((_K3, 16), jnp.float32),
            pltpu.VMEM((_K3, 16), jnp.float32),
            pltpu.VMEM((_K3 * 8,), jnp.float32),
            pltpu.VMEM((_K3 * 8 // 128, 128), jnp.int32),
            pltpu.VMEM((5120,), jnp.float32),
            pltpu.VMEM_SHARED((8 * _NP,), jnp.float32),
        ],
    )
    return f(srcp, dstp, aet, asrc, adst)


def _out_body(srcp, dstp, exp_in, h_in, den_in, offs, bias,
              xout,
              src_c, drel_c, ex_c, rows, fin, zbuf, bvm, bias_c,
              dv0, dv1, inv_c, acc_sp, *, last):
    c = lax.axis_index("c")
    s = lax.axis_index("s")
    zero16 = jnp.zeros((16,), jnp.float32)

    def _z(i, _):
        zbuf[i >> 6, pl.ds((i & 63) * 16, 16)] = zero16
        return 0
    lax.fori_loop(0, 1024, _z, 0)
    pltpu.sync_copy(offs, bvm)
    pltpu.sync_copy(bias, bias_c)
    bv0 = bvm[pl.ds(0, 16)]
    bv1 = bvm[pl.ds(16, 16)]
    bv2 = bvm[pl.ds(32, 16)]

    _offvals = ([bv0[k] for k in range(16)]
                + [bv1[k] for k in range(16)]
                + [bv2[k] for k in range(_NB + 1 - 32)])

    def _get_off(b):
        v = _offvals[0]
        for k in range(1, _NB + 1):
            v = jnp.where(b == k, _offvals[k], v)
        return v

    def _block(bb, _):
        b = bb * 2 + c
        lo = pl.multiple_of(_get_off(b), 1024)
        hi = pl.multiple_of(_get_off(b + 1), 1024)

        pltpu.sync_copy(zbuf, acc_sp.at[pl.ds(s * 16, 16)])

        @pl.when(s == 0)
        def _():
            pltpu.sync_copy(zbuf.at[pl.ds(0, 1)], acc_sp.at[pl.ds(_BS, 1)])
        plsc.subcore_barrier()

        def _chunk(i, _):
            ch = pl.multiple_of(lo + (i * 16 + s) * _K4, 64)
            pltpu.sync_copy(srcp.at[pl.ds(ch, _K4)], src_c)
            pltpu.sync_copy(exp_in.at[pl.ds(ch * 8, _K4 * 8)], ex_c)
            pltpu.sync_copy(h_in.at[src_c], rows)
            pltpu.sync_copy(dstp.at[pl.ds(ch, _K4)], drel_c)
            for g in range(_K4 // 16):
                dstv = drel_c[pl.ds(g * 16, 16)]
                rel = dstv - b * _BS
                inb = (rel >= 0) & (rel < _BS)
                drel_c[pl.ds(g * 16, 16)] = jnp.where(inb, rel, _BS)

            def _scale(e, _):
                exv = ex_c[pl.ds((e >> 1) * 16, 16)]
                base = (e & 1) * 8
                for h in range(HEADS):
                    sp = jnp.take(exv, _spl(0) + (base + h))
                    for v in range(C // 16):
                        off = h * C + v * 16
                        rows[e, pl.ds(off, 16)] = rows[e, pl.ds(off, 16)] * sp
                return 0
            lax.fori_loop(0, _K4, _scale, 0)
            pltpu.sync_copy(rows, acc_sp.at[drel_c], add=True)
            return 0
        lax.fori_loop(0, (hi - lo) // (16 * _K4), _chunk, 0)
        plsc.subcore_barrier()

        rcount = jnp.minimum(_BS, N - b * _BS)

        def _fingrp(q, _):
            r0 = s * 16 + q * 0

            @pl.when(r0 < rcount)
            def _():
                row0 = b * _BS + r0
                pltpu.sync_copy(acc_sp.at[pl.ds(r0, 16)], fin)
                pltpu.sync_copy(den_in.at[pl.ds(row0 * 8, 128)], dv0)
                pltpu.sync_copy(den_in.at[pl.ds(8 * _NP + row0 * 8, 128)],
                                dv1)
                for k in range(8):
                    d = dv0[pl.ds(k * 16, 16)] + dv1[pl.ds(k * 16, 16)]
                    inv_c[pl.ds(k * 16, 16)] = 1.0 / (d + 1e-16)

                def _row(r, _):
                    iv = inv_c[pl.ds((r >> 1) * 16, 16)]
                    base = (r & 1) * 8
                    if last:
                        for v in range(C // 16):
                            acc = bias_c[pl.ds(v * 16, 16)]
                            for h in range(HEADS):
                                sp = jnp.take(iv, _spl(0) + (base + h)) * 0.125
                                acc = acc + fin[r, pl.ds(h * C + v * 16, 16)] * sp
                            fin[r, pl.ds(v * 16, 16)] = acc
                    else:
                        for h in range(HEADS):
                            sp = jnp.take(iv, _spl(0) + (base + h))
                            for v in range(C // 16):
                                off = h * C + v * 16
                                fin[r, pl.ds(off, 16)] = (
                                    fin[r, pl.ds(off, 16)] * sp
                                    + bias_c[pl.ds(off, 16)])
                    return 0
                lax.fori_loop(0, 16, _row, 0)
                if last:
                    pltpu.sync_copy(fin.at[:, pl.ds(0, C)],
                                    xout.at[pl.ds(row0, 16)])
                else:
                    pltpu.sync_copy(fin, xout.at[pl.ds(row0, 16)])
            return 0
        lax.fori_loop(0, 1, _fingrp, 0)
        plsc.subcore_barrier()
        return 0

    lax.fori_loop(0, _NB // 2, _block, 0)


def _out_pass(srcp, dstp, exp_e, h, den, offs, bias, last):
    odim = C if last else HC
    f = pl.kernel(
        functools.partial(_out_body, last=last),
        out_type=jax.ShapeDtypeStruct((N, odim), jnp.float32),
        mesh=_sc_mesh,
        compiler_params=_sc_params,
        scratch_types=[
            pltpu.VMEM((_K4,), jnp.int32),
            pltpu.VMEM((_K4,), jnp.int32),
            pltpu.VMEM((_K4 * 8,), jnp.float32),
            pltpu.VMEM((_K4, HC), jnp.float32),
            pltpu.VMEM((16, HC), jnp.float32),
            pltpu.VMEM((16, HC), jnp.float32),
            pltpu.VMEM((48,), jnp.int32),
            pltpu.VMEM((HC,), jnp.float32),
            pltpu.VMEM((128,), jnp.float32),
            pltpu.VMEM((128,), jnp.float32),
            pltpu.VMEM((128,), jnp.float32),
            pltpu.VMEM_SHARED((_BS + 1, HC), jnp.float32),
        ],
    )
    return f(srcp, dstp, exp_e, h, den, offs, bias)


def _bucket_edges(src, dst, edge_attr):
    blk = dst >> 8
    counts = jnp.bincount(blk, length=_NB)
    caps = ((counts + 1023) // 1024) * 1024
    offs = jnp.concatenate([jnp.zeros((1,), counts.dtype),
                            jnp.cumsum(caps)]).astype(jnp.int32)
    order = jnp.argsort(blk, stable=True)
    sblk = blk[order]
    cstart = jnp.concatenate([jnp.zeros((1,), counts.dtype),
                              jnp.cumsum(counts)])[:-1]
    pos = offs[sblk] + (jnp.arange(E, dtype=jnp.int32)
                        - cstart[sblk].astype(jnp.int32))
    srcp = jnp.zeros((_EP,), jnp.int32).at[pos].set(src[order])
    dstp = jnp.full((_EP,), _DUMMY, jnp.int32).at[pos].set(dst[order])
    eap = jnp.zeros((_EP, EDGE_DIM), jnp.float32).at[pos].set(edge_attr[order])
    offs32 = jnp.zeros((48,), jnp.int32).at[:_NB + 1].set(offs)
    return srcp, dstp, eap, offs32


def kernel(x, edge_index, edge_attr,
           W1, We1, as1, ad1, ae1, b1,
           W2, We2, as2, ad2, ae2, b2,
           W3, We3, as3, ad3, ae3, b3):
    src = edge_index[0]
    dst = edge_index[1]
    srcp, dstp, eap, offs32 = _bucket_edges(src, dst, edge_attr)
    srcp2 = srcp.reshape(_EP // 128, 128)
    dstp2 = dstp.reshape(_EP // 128, 128)

    Wc1, Ve1 = _fold_weights(W1, We1, as1, ad1, ae1)
    Wc2, Ve2 = _fold_weights(W2, We2, as2, ad2, ae2)
    Wc3, Ve3 = _fold_weights(W3, We3, as3, ad3, ae3)
    Vcat = jnp.concatenate([Ve1, Ve2, Ve3], axis=1)

    ae_all = _edge_logits(eap, Vcat)

    xin = x
    for li, (Wc, bias) in enumerate(((Wc1, b1), (Wc2, b2), (Wc3, b3))):
        h_all = _node_matmul(xin, Wc, relu=(li > 0))
        h = h_all[:, :HC]
        asrc = h_all[:, HC:HC + 16]
        adst = jnp.pad(h_all[:, HC + 16:HC + 32], ((0, _NP - N), (0, 0)))
        aet = lax.dynamic_slice_in_dim(ae_all, li * 16, 16, axis=1)
        exp_e, den = _alpha_pass(srcp2, dstp2, aet, asrc, adst)
        bias_p = bias if li < 2 else jnp.pad(bias, (0, HC - C))
        xin = _out_pass(srcp, dstp, exp_e, h, den, offs32, bias_p,
                        last=(li == 2))
    return xin


# speedup vs baseline: 5.7360x; 1.0411x over previous
import functools

import jax
import jax.numpy as jnp
from jax import lax
from jax.experimental import pallas as pl
from jax.experimental.pallas import tpu as pltpu
from jax.experimental.pallas import tpu_sc as plsc

N = 10000
E = 320000
HEADS = 8
C = 128
HC = HEADS * C
EDGE_DIM = 32

_M_TILE = 400
_E_TILE = 2048

_NC = 2
_NS = 16
_NW = _NC * _NS

_BS = 256
_NB = 40
_EP = 393216
_EPT = _EP // _NW
_K3 = 1024
_NCH3 = _EPT // _K3
_K4 = 32
_NP = 10240
_DUMMY = N

_sc_mesh = plsc.VectorSubcoreMesh(
    core_axis_name="c", subcore_axis_name="s", num_cores=_NC, num_subcores=_NS)
_sc_params = pltpu.CompilerParams(use_tc_tiling_on_sc=False)


def _i16():
    return lax.broadcasted_iota(jnp.int32, (16,), 0)


def _spl(v):
    return jnp.full((16,), v, jnp.int32)


def _comb8(a, b):
    i = _i16()
    return jnp.where(i < 8, a, jnp.take(b, i & 7))


def _fold_weights(W, We, a_s, a_d, a_e):
    d = W.shape[0]
    Was = (W.reshape(d, HEADS, C) * a_s[None]).sum(-1)
    Wad = (W.reshape(d, HEADS, C) * a_d[None]).sum(-1)
    Ve = (We.reshape(EDGE_DIM, HEADS, C) * a_e[None]).sum(-1)
    z8 = jnp.zeros((d, 8), jnp.float32)
    pad = jnp.zeros((d, 1152 - HC - 32), jnp.float32)
    Wcat = jnp.concatenate([W, Was, z8, Wad, z8, pad], axis=1)
    Ve16 = jnp.concatenate([Ve, jnp.zeros((EDGE_DIM, 8), jnp.float32)], axis=1)
    return Wcat, Ve16


def _mm_body(x_ref, w_ref, o_ref, *, relu):
    xb = x_ref[...]
    if relu:
        xb = jnp.maximum(xb, 0.0)
    o_ref[...] = jnp.dot(xb, w_ref[...], preferred_element_type=jnp.float32)


def _node_matmul(xin, Wcat, relu):
    K = xin.shape[1]
    return pl.pallas_call(
        functools.partial(_mm_body, relu=relu),
        grid=(N // _M_TILE,),
        in_specs=[
            pl.BlockSpec((_M_TILE, K), lambda i: (i, 0)),
            pl.BlockSpec((K, 1152), lambda i: (0, 0)),
        ],
        out_specs=pl.BlockSpec((_M_TILE, 1152), lambda i: (i, 0)),
        out_shape=jax.ShapeDtypeStruct((N, 1152), jnp.float32),
    )(xin, Wcat)


def _ae_body(ea_ref, v_ref, o_ref):
    o_ref[...] = jnp.dot(ea_ref[...], v_ref[...],
                         preferred_element_type=jnp.float32)


def _edge_logits(eap, Vcat):
    return pl.pallas_call(
        _ae_body,
        grid=(_EP // _E_TILE,),
        in_specs=[
            pl.BlockSpec((_E_TILE, EDGE_DIM), lambda i: (i, 0)),
            pl.BlockSpec((EDGE_DIM, 48), lambda i: (0, 0)),
        ],
        out_specs=pl.BlockSpec((_E_TILE, 48), lambda i: (i, 0)),
        out_shape=jax.ShapeDtypeStruct((_EP, 48), jnp.float32),
    )(eap, Vcat)


def _alpha_body(srcp, dstp, aet, asrc, adst,
                exp_out, den_out,
                src_c, dst_c, ae_c, asg, adg, ex_s, didx_c, zrow, den_sp):
    c = lax.axis_index("c")
    s = lax.axis_index("s")
    zero16 = jnp.zeros((16,), jnp.float32)

    def _z(i, _):
        zrow[pl.ds(i * 16, 16)] = zero16
        return 0
    lax.fori_loop(0, 320, _z, 0)
    pltpu.sync_copy(zrow, den_sp.at[pl.ds(s * 5120, 5120)])
    plsc.subcore_barrier()

    gbase = c * (_EP // _NC) + s * _EPT
    i16 = _i16()
    lane7 = i16 & 7

    def _chunk(i, _):
        b = gbase + i * _K3
        br = b // 128
        pltpu.sync_copy(srcp.at[pl.ds(br, 8)], src_c)
        pltpu.sync_copy(dstp.at[pl.ds(br, 8)], dst_c)
        pltpu.sync_copy(aet.at[pl.ds(b, _K3)], ae_c)
        for t in range(8):
            pltpu.sync_copy(asrc.at[src_c.at[t]], asg.at[pl.ds(t * 128, 128)])
        for t in range(8):
            pltpu.sync_copy(adst.at[dst_c.at[t]], adg.at[pl.ds(t * 128, 128)])

        def _pair(m, _):
            av = _comb8(asg[2 * m], asg[2 * m + 1])
            bv = _comb8(adg[2 * m], adg[2 * m + 1])
            ev = _comb8(ae_c[2 * m], ae_c[2 * m + 1])
            al = av + bv + ev
            al = jnp.where(al > 0, al, al * 0.2)
            exv = jnp.exp(al)
            ex_s[pl.ds(m * 16, 16)] = exv
            dstv = dst_c[m >> 6, pl.ds(((m >> 3) & 7) * 16, 16)]
            e0 = (m & 7) * 2
            d0 = jnp.take(dstv, _spl(0) + e0)
            d1 = jnp.take(dstv, _spl(0) + (e0 + 1))
            didx = jnp.where(i16 < 8, d0 * 8 + i16, d1 * 8 + lane7)
            didx_c[m >> 3, pl.ds((m & 7) * 16, 16)] = didx
            return 0
        lax.fori_loop(0, _K3 // 2, _pair, 0)

        def _scat(t, _):
            pltpu.sync_copy(ex_s.at[pl.ds(t * 128, 128)],
                            den_sp.at[didx_c.at[t]], add=True)
            return 0
        lax.fori_loop(0, _K3 * 8 // 128, _scat, 0)
        pltpu.sync_copy(ex_s, exp_out.at[pl.ds(b * 8, _K3 * 8)])
        return 0

    lax.fori_loop(0, _NCH3, _chunk, 0)
    plsc.subcore_barrier()

    @pl.when(s == 0)
    def _():
        pltpu.sync_copy(den_sp, den_out.at[pl.ds(c * 8 * _NP, 8 * _NP)])


def _alpha_pass(srcp, dstp, aet, asrc, adst):
    f = pl.kernel(
        _alpha_body,
        out_type=(jax.ShapeDtypeStruct((_EP * 8,), jnp.float32),
                  jax.ShapeDtypeStruct((2 * 8 * _NP,), jnp.float32)),
        mesh=_sc_mesh,
        compiler_params=_sc_params,
        scratch_types=[
            pltpu.VMEM((8, 128), jnp.int32),
            pltpu.VMEM((8, 128), jnp.int32),
            pltpu.VMEM((_K3, 16), jnp.float32),
            pltpu.VMEM((_K3, 16), jnp.float32),
            pltpu.VMEM((_K3, 16), jnp.float32),
            pltpu.VMEM((_K3 * 8,), jnp.float32),
            pltpu.VMEM((_K3 * 8 // 128, 128), jnp.int32),
            pltpu.VMEM((5120,), jnp.float32),
            pltpu.VMEM_SHARED((8 * _NP,), jnp.float32),
        ],
    )
    return f(srcp, dstp, aet, asrc, adst)


def _out_body(srcp, dstp, exp_in, h_in, den_in, offs, bias,
              xout,
              src_c0, src_c1, drel_c, ex_c, rows0, rows1, fin, zbuf, bvm,
              bias_c, dv0, dv1, inv_c, sem0, sem1, acc_sp, *, last):
    c = lax.axis_index("c")
    s = lax.axis_index("s")
    zero16 = jnp.zeros((16,), jnp.float32)

    def _z(i, _):
        zbuf[i >> 6, pl.ds((i & 63) * 16, 16)] = zero16
        return 0
    lax.fori_loop(0, 512, _z, 0)
    pltpu.sync_copy(offs, bvm)
    pltpu.sync_copy(bias, bias_c)
    bv0 = bvm[pl.ds(0, 16)]
    bv1 = bvm[pl.ds(16, 16)]
    bv2 = bvm[pl.ds(32, 16)]

    _offvals = ([bv0[k] for k in range(16)]
                + [bv1[k] for k in range(16)]
                + [bv2[k] for k in range(_NB + 1 - 32)])

    def _get_off(b):
        v = _offvals[0]
        for k in range(1, _NB + 1):
            v = jnp.where(b == k, _offvals[k], v)
        return v

    def _block(bb, _):
        b = bb * 2 + c
        lo = pl.multiple_of(_get_off(b), 1024)
        hi = pl.multiple_of(_get_off(b + 1), 1024)

        pltpu.sync_copy(zbuf, acc_sp.at[pl.ds(s * 16, 8)])
        pltpu.sync_copy(zbuf, acc_sp.at[pl.ds(s * 16 + 8, 8)])

        @pl.when(s == 0)
        def _():
            pltpu.sync_copy(zbuf.at[pl.ds(0, 1)], acc_sp.at[pl.ds(_BS, 1)])
        plsc.subcore_barrier()

        def _ch_off(k):
            ch = lo + (k * 16 + s) * _K4
            return pl.multiple_of(jnp.minimum(ch, _EP - _K4), _K4)

        def _start(k, src_b, rows_b, sem):
            ch = _ch_off(k)
            pltpu.sync_copy(srcp.at[pl.ds(ch, _K4)], src_b)
            pltpu.async_copy(h_in.at[src_b], rows_b, sem)

        def _do(k, src_b, rows_b, sem):
            pltpu.make_async_copy(h_in.at[src_b], rows_b, sem).wait()
            ch = _ch_off(k)
            pltpu.sync_copy(exp_in.at[pl.ds(ch * 8, _K4 * 8)], ex_c)
            pltpu.sync_copy(dstp.at[pl.ds(ch, _K4)], drel_c)
            for g in range(_K4 // 16):
                dstv = drel_c[pl.ds(g * 16, 16)]
                rel = dstv - b * _BS
                inb = (rel >= 0) & (rel < _BS)
                drel_c[pl.ds(g * 16, 16)] = jnp.where(inb, rel, _BS)

            def _scale(e, _):
                exv = ex_c[pl.ds((e >> 1) * 16, 16)]
                base = (e & 1) * 8
                for h in range(HEADS):
                    sp = jnp.take(exv, _spl(0) + (base + h))
                    for v in range(C // 16):
                        off = h * C + v * 16
                        rows_b[e, pl.ds(off, 16)] = (
                            rows_b[e, pl.ds(off, 16)] * sp)
                return 0
            lax.fori_loop(0, _K4, _scale, 0)
            pltpu.sync_copy(rows_b, acc_sp.at[drel_c], add=True)

        _start(0, src_c0, rows0, sem0)
        _start(1, src_c1, rows1, sem1)

        def _piter(i2, _):
            k0 = i2 * 2
            _do(k0, src_c0, rows0, sem0)
            _start(k0 + 2, src_c0, rows0, sem0)
            _do(k0 + 1, src_c1, rows1, sem1)
            _start(k0 + 3, src_c1, rows1, sem1)
            return 0
        lax.fori_loop(0, (hi - lo) // (32 * _K4), _piter, 0)
        pltpu.make_async_copy(h_in.at[src_c0], rows0, sem0).wait()
        pltpu.make_async_copy(h_in.at[src_c1], rows1, sem1).wait()
        plsc.subcore_barrier()

        rcount = jnp.minimum(_BS, N - b * _BS)

        def _fingrp(q, _):
            r0 = s * 16 + q * 0

            @pl.when(r0 < rcount)
            def _():
                row0 = b * _BS + r0
                pltpu.sync_copy(acc_sp.at[pl.ds(r0, 16)], fin)
                pltpu.sync_copy(den_in.at[pl.ds(row0 * 8, 128)], dv0)
                pltpu.sync_copy(den_in.at[pl.ds(8 * _NP + row0 * 8, 128)],
                                dv1)
                for k in range(8):
                    d = dv0[pl.ds(k * 16, 16)] + dv1[pl.ds(k * 16, 16)]
                    inv_c[pl.ds(k * 16, 16)] = 1.0 / (d + 1e-16)

                def _row(r, _):
                    iv = inv_c[pl.ds((r >> 1) * 16, 16)]
                    base = (r & 1) * 8
                    if last:
                        for v in range(C // 16):
                            acc = bias_c[pl.ds(v * 16, 16)]
                            for h in range(HEADS):
                                sp = jnp.take(iv, _spl(0) + (base + h)) * 0.125
                                acc = acc + fin[r, pl.ds(h * C + v * 16, 16)] * sp
                            fin[r, pl.ds(v * 16, 16)] = acc
                    else:
                        for h in range(HEADS):
                            sp = jnp.take(iv, _spl(0) + (base + h))
                            for v in range(C // 16):
                                off = h * C + v * 16
                                fin[r, pl.ds(off, 16)] = (
                                    fin[r, pl.ds(off, 16)] * sp
                                    + bias_c[pl.ds(off, 16)])
                    return 0
                lax.fori_loop(0, 16, _row, 0)
                if last:
                    pltpu.sync_copy(fin.at[:, pl.ds(0, C)],
                                    xout.at[pl.ds(row0, 16)])
                else:
                    pltpu.sync_copy(fin, xout.at[pl.ds(row0, 16)])
            return 0
        lax.fori_loop(0, 1, _fingrp, 0)
        plsc.subcore_barrier()
        return 0

    lax.fori_loop(0, _NB // 2, _block, 0)


def _out_pass(srcp, dstp, exp_e, h, den, offs, bias, last):
    odim = C if last else HC
    f = pl.kernel(
        functools.partial(_out_body, last=last),
        out_type=jax.ShapeDtypeStruct((N, odim), jnp.float32),
        mesh=_sc_mesh,
        compiler_params=_sc_params,
        scratch_types=[
            pltpu.VMEM((_K4,), jnp.int32),
            pltpu.VMEM((_K4,), jnp.int32),
            pltpu.VMEM((_K4,), jnp.int32),
            pltpu.VMEM((_K4 * 8,), jnp.float32),
            pltpu.VMEM((_K4, HC), jnp.float32),
            pltpu.VMEM((_K4, HC), jnp.float32),
            pltpu.VMEM((16, HC), jnp.float32),
            pltpu.VMEM((8, HC), jnp.float32),
            pltpu.VMEM((48,), jnp.int32),
            pltpu.VMEM((HC,), jnp.float32),
            pltpu.VMEM((128,), jnp.float32),
            pltpu.VMEM((128,), jnp.float32),
            pltpu.VMEM((128,), jnp.float32),
            pltpu.SemaphoreType.DMA,
            pltpu.SemaphoreType.DMA,
            pltpu.VMEM_SHARED((_BS + 1, HC), jnp.float32),
        ],
    )
    return f(srcp, dstp, exp_e, h, den, offs, bias)


def _bucket_edges(src, dst, edge_attr):
    blk = dst >> 8
    counts = jnp.bincount(blk, length=_NB)
    caps = ((counts + 1023) // 1024) * 1024
    offs = jnp.concatenate([jnp.zeros((1,), counts.dtype),
                            jnp.cumsum(caps)]).astype(jnp.int32)
    order = jnp.argsort(blk, stable=True)
    sblk = blk[order]
    cstart = jnp.concatenate([jnp.zeros((1,), counts.dtype),
                              jnp.cumsum(counts)])[:-1]
    pos = offs[sblk] + (jnp.arange(E, dtype=jnp.int32)
                        - cstart[sblk].astype(jnp.int32))
    srcp = jnp.zeros((_EP,), jnp.int32).at[pos].set(src[order])
    dstp = jnp.full((_EP,), _DUMMY, jnp.int32).at[pos].set(dst[order])
    eap = jnp.zeros((_EP, EDGE_DIM), jnp.float32).at[pos].set(edge_attr[order])
    offs32 = jnp.zeros((48,), jnp.int32).at[:_NB + 1].set(offs)
    return srcp, dstp, eap, offs32


def kernel(x, edge_index, edge_attr,
           W1, We1, as1, ad1, ae1, b1,
           W2, We2, as2, ad2, ae2, b2,
           W3, We3, as3, ad3, ae3, b3):
    src = edge_index[0]
    dst = edge_index[1]
    srcp, dstp, eap, offs32 = _bucket_edges(src, dst, edge_attr)
    srcp2 = srcp.reshape(_EP // 128, 128)
    dstp2 = dstp.reshape(_EP // 128, 128)

    Wc1, Ve1 = _fold_weights(W1, We1, as1, ad1, ae1)
    Wc2, Ve2 = _fold_weights(W2, We2, as2, ad2, ae2)
    Wc3, Ve3 = _fold_weights(W3, We3, as3, ad3, ae3)
    Vcat = jnp.concatenate([Ve1, Ve2, Ve3], axis=1)

    ae_all = _edge_logits(eap, Vcat)

    xin = x
    for li, (Wc, bias) in enumerate(((Wc1, b1), (Wc2, b2), (Wc3, b3))):
        h_all = _node_matmul(xin, Wc, relu=(li > 0))
        h = h_all[:, :HC]
        asrc = h_all[:, HC:HC + 16]
        adst = jnp.pad(h_all[:, HC + 16:HC + 32], ((0, _NP - N), (0, 0)))
        aet = lax.dynamic_slice_in_dim(ae_all, li * 16, 16, axis=1)
        exp_e, den = _alpha_pass(srcp2, dstp2, aet, asrc, adst)
        bias_p = bias if li < 2 else jnp.pad(bias, (0, HC - C))
        xin = _out_pass(srcp, dstp, exp_e, h, den, offs32, bias_p,
                        last=(li == 2))
    return xin


# speedup vs baseline: 5.8202x; 1.0147x over previous
import functools

import jax
import jax.numpy as jnp
from jax import lax
from jax.experimental import pallas as pl
from jax.experimental.pallas import tpu as pltpu
from jax.experimental.pallas import tpu_sc as plsc

N = 10000
E = 320000
HEADS = 8
C = 128
HC = HEADS * C
EDGE_DIM = 32

_M_TILE = 400
_E_TILE = 2048

_NC = 2
_NS = 16
_NW = _NC * _NS

_BS = 256
_NB = 40
_EP = 393216
_EPT = _EP // _NW
_K3 = 1024
_NCH3 = _EPT // _K3
_K4 = 32
_NP = 10240
_DUMMY = N

_sc_mesh = plsc.VectorSubcoreMesh(
    core_axis_name="c", subcore_axis_name="s", num_cores=_NC, num_subcores=_NS)
_sc_params = pltpu.CompilerParams(use_tc_tiling_on_sc=False)


def _i16():
    return lax.broadcasted_iota(jnp.int32, (16,), 0)


def _spl(v):
    return jnp.full((16,), v, jnp.int32)


def _comb8(a, b):
    i = _i16()
    return jnp.where(i < 8, a, jnp.take(b, i & 7))


def _fold_weights(W, We, a_s, a_d, a_e):
    d = W.shape[0]
    Was = (W.reshape(d, HEADS, C) * a_s[None]).sum(-1)
    Wad = (W.reshape(d, HEADS, C) * a_d[None]).sum(-1)
    Ve = (We.reshape(EDGE_DIM, HEADS, C) * a_e[None]).sum(-1)
    z8 = jnp.zeros((d, 8), jnp.float32)
    pad = jnp.zeros((d, 1152 - HC - 32), jnp.float32)
    Wcat = jnp.concatenate([W, Was, z8, Wad, z8, pad], axis=1)
    Ve16 = jnp.concatenate([Ve, jnp.zeros((EDGE_DIM, 8), jnp.float32)], axis=1)
    return Wcat, Ve16


def _mm_body(x_ref, w_ref, o_ref, *, relu):
    xb = x_ref[...]
    if relu:
        xb = jnp.maximum(xb, 0.0)
    o_ref[...] = jnp.dot(xb, w_ref[...], preferred_element_type=jnp.float32)


def _node_matmul(xin, Wcat, relu):
    K = xin.shape[1]
    return pl.pallas_call(
        functools.partial(_mm_body, relu=relu),
        grid=(N // _M_TILE,),
        in_specs=[
            pl.BlockSpec((_M_TILE, K), lambda i: (i, 0)),
            pl.BlockSpec((K, 1152), lambda i: (0, 0)),
        ],
        out_specs=pl.BlockSpec((_M_TILE, 1152), lambda i: (i, 0)),
        out_shape=jax.ShapeDtypeStruct((N, 1152), jnp.float32),
    )(xin, Wcat)


def _ae_body(ea_ref, v_ref, o1_ref, o2_ref, o3_ref):
    d = jnp.dot(ea_ref[...], v_ref[...], preferred_element_type=jnp.float32)
    o1_ref[...] = d[:, 0:16]
    o2_ref[...] = d[:, 16:32]
    o3_ref[...] = d[:, 32:48]


def _edge_logits(eap, Vcat):
    spec = pl.BlockSpec((_E_TILE, 16), lambda i: (i, 0))
    shp = jax.ShapeDtypeStruct((_EP, 16), jnp.float32)
    return pl.pallas_call(
        _ae_body,
        grid=(_EP // _E_TILE,),
        in_specs=[
            pl.BlockSpec((_E_TILE, EDGE_DIM), lambda i: (i, 0)),
            pl.BlockSpec((EDGE_DIM, 48), lambda i: (0, 0)),
        ],
        out_specs=(spec, spec, spec),
        out_shape=(shp, shp, shp),
    )(eap, Vcat)


def _alpha_body(srcp, dstp, aet, asrc, adst,
                exp_out, den_out,
                src_c, dst_c, ae_c, asg, adg, ex_s, didx_c, zrow, den_sp):
    c = lax.axis_index("c")
    s = lax.axis_index("s")
    zero16 = jnp.zeros((16,), jnp.float32)

    def _z(i, _):
        zrow[pl.ds(i * 16, 16)] = zero16
        return 0
    lax.fori_loop(0, 320, _z, 0)
    pltpu.sync_copy(zrow, den_sp.at[pl.ds(s * 5120, 5120)])
    plsc.subcore_barrier()

    gbase = c * (_EP // _NC) + s * _EPT
    i16 = _i16()
    lane7 = i16 & 7

    def _chunk(i, _):
        b = gbase + i * _K3
        br = b // 128
        pltpu.sync_copy(srcp.at[pl.ds(br, 8)], src_c)
        pltpu.sync_copy(dstp.at[pl.ds(br, 8)], dst_c)
        pltpu.sync_copy(aet.at[pl.ds(b, _K3)], ae_c)
        for t in range(8):
            pltpu.sync_copy(asrc.at[src_c.at[t]], asg.at[pl.ds(t * 128, 128)])
        for t in range(8):
            pltpu.sync_copy(adst.at[dst_c.at[t]], adg.at[pl.ds(t * 128, 128)])

        def _pair(m, _):
            av = _comb8(asg[2 * m], asg[2 * m + 1])
            bv = _comb8(adg[2 * m], adg[2 * m + 1])
            ev = _comb8(ae_c[2 * m], ae_c[2 * m + 1])
            al = av + bv + ev
            al = jnp.where(al > 0, al, al * 0.2)
            exv = jnp.exp(al)
            ex_s[pl.ds(m * 16, 16)] = exv
            dstv = dst_c[m >> 6, pl.ds(((m >> 3) & 7) * 16, 16)]
            e0 = (m & 7) * 2
            d0 = jnp.take(dstv, _spl(0) + e0)
            d1 = jnp.take(dstv, _spl(0) + (e0 + 1))
            didx = jnp.where(i16 < 8, d0 * 8 + i16, d1 * 8 + lane7)
            didx_c[m >> 3, pl.ds((m & 7) * 16, 16)] = didx
            return 0
        lax.fori_loop(0, _K3 // 2, _pair, 0)

        def _scat(t, _):
            pltpu.sync_copy(ex_s.at[pl.ds(t * 128, 128)],
                            den_sp.at[didx_c.at[t]], add=True)
            return 0
        lax.fori_loop(0, _K3 * 8 // 128, _scat, 0)
        pltpu.sync_copy(ex_s, exp_out.at[pl.ds(b * 8, _K3 * 8)])
        return 0

    lax.fori_loop(0, _NCH3, _chunk, 0)
    plsc.subcore_barrier()

    @pl.when(s == 0)
    def _():
        pltpu.sync_copy(den_sp, den_out.at[pl.ds(c * 8 * _NP, 8 * _NP)])


def _alpha_pass(srcp, dstp, aet, asrc, adst):
    f = pl.kernel(
        _alpha_body,
        out_type=(jax.ShapeDtypeStruct((_EP * 8,), jnp.float32),
                  jax.ShapeDtypeStruct((2 * 8 * _NP,), jnp.float32)),
        mesh=_sc_mesh,
        compiler_params=_sc_params,
        scratch_types=[
            pltpu.VMEM((8, 128), jnp.int32),
            pltpu.VMEM((8, 128), jnp.int32),
            pltpu.VMEM((_K3, 16), jnp.float32),
            pltpu.VMEM((_K3, 16), jnp.float32),
            pltpu.VMEM((_K3, 16), jnp.float32),
            pltpu.VMEM((_K3 * 8,), jnp.float32),
            pltpu.VMEM((_K3 * 8 // 128, 128), jnp.int32),
            pltpu.VMEM((5120,), jnp.float32),
            pltpu.VMEM_SHARED((8 * _NP,), jnp.float32),
        ],
    )
    return f(srcp, dstp, aet, asrc, adst)


def _out_body(srcp, dstp, exp_in, h_in, den_in, offs, bias,
              xout,
              src_c0, src_c1, drel_c, ex_c, rows0, rows1, fin, zbuf, bvm,
              bias_c, dv0, dv1, inv_c, sem0, sem1, acc_sp, *, last):
    c = lax.axis_index("c")
    s = lax.axis_index("s")
    zero16 = jnp.zeros((16,), jnp.float32)

    def _z(i, _):
        zbuf[i >> 6, pl.ds((i & 63) * 16, 16)] = zero16
        return 0
    lax.fori_loop(0, 512, _z, 0)
    pltpu.sync_copy(offs, bvm)
    pltpu.sync_copy(bias, bias_c)
    bv0 = bvm[pl.ds(0, 16)]
    bv1 = bvm[pl.ds(16, 16)]
    bv2 = bvm[pl.ds(32, 16)]

    _offvals = ([bv0[k] for k in range(16)]
                + [bv1[k] for k in range(16)]
                + [bv2[k] for k in range(_NB + 1 - 32)])

    def _get_off(b):
        v = _offvals[0]
        for k in range(1, _NB + 1):
            v = jnp.where(b == k, _offvals[k], v)
        return v

    def _block(bb, _):
        b = bb * 2 + c
        lo = pl.multiple_of(_get_off(b), 1024)
        hi = pl.multiple_of(_get_off(b + 1), 1024)

        pltpu.sync_copy(zbuf, acc_sp.at[pl.ds(s * 16, 8)])
        pltpu.sync_copy(zbuf, acc_sp.at[pl.ds(s * 16 + 8, 8)])

        @pl.when(s == 0)
        def _():
            pltpu.sync_copy(zbuf.at[pl.ds(0, 1)], acc_sp.at[pl.ds(_BS, 1)])
        plsc.subcore_barrier()

        def _ch_off(k):
            ch = lo + (k * 16 + s) * _K4
            return pl.multiple_of(jnp.minimum(ch, _EP - _K4), _K4)

        def _start(k, src_b, rows_b, sem):
            ch = _ch_off(k)
            pltpu.sync_copy(srcp.at[pl.ds(ch, _K4)], src_b)
            pltpu.async_copy(h_in.at[src_b], rows_b, sem)

        def _do(k, src_b, rows_b, sem):
            pltpu.make_async_copy(h_in.at[src_b], rows_b, sem).wait()
            ch = _ch_off(k)
            pltpu.sync_copy(exp_in.at[pl.ds(ch * 8, _K4 * 8)], ex_c)
            pltpu.sync_copy(dstp.at[pl.ds(ch, _K4)], drel_c)
            for g in range(_K4 // 16):
                dstv = drel_c[pl.ds(g * 16, 16)]
                rel = dstv - b * _BS
                inb = (rel >= 0) & (rel < _BS)
                drel_c[pl.ds(g * 16, 16)] = jnp.where(inb, rel, _BS)

            def _scale(e, _):
                exv = ex_c[pl.ds((e >> 1) * 16, 16)]
                base = (e & 1) * 8
                for h in range(HEADS):
                    sp = jnp.take(exv, _spl(0) + (base + h))
                    for v in range(C // 16):
                        off = h * C + v * 16
                        rows_b[e, pl.ds(off, 16)] = (
                            rows_b[e, pl.ds(off, 16)] * sp)
                return 0
            lax.fori_loop(0, _K4, _scale, 0)
            pltpu.sync_copy(rows_b, acc_sp.at[drel_c], add=True)

        _start(0, src_c0, rows0, sem0)
        _start(1, src_c1, rows1, sem1)

        def _piter(i2, _):
            k0 = i2 * 2
            _do(k0, src_c0, rows0, sem0)
            _start(k0 + 2, src_c0, rows0, sem0)
            _do(k0 + 1, src_c1, rows1, sem1)
            _start(k0 + 3, src_c1, rows1, sem1)
            return 0
        lax.fori_loop(0, (hi - lo) // (32 * _K4), _piter, 0)
        pltpu.make_async_copy(h_in.at[src_c0], rows0, sem0).wait()
        pltpu.make_async_copy(h_in.at[src_c1], rows1, sem1).wait()
        plsc.subcore_barrier()

        rcount = jnp.minimum(_BS, N - b * _BS)

        def _fingrp(q, _):
            r0 = s * 16 + q * 0

            @pl.when(r0 < rcount)
            def _():
                row0 = b * _BS + r0
                pltpu.sync_copy(acc_sp.at[pl.ds(r0, 16)], fin)
                pltpu.sync_copy(den_in.at[pl.ds(row0 * 8, 128)], dv0)
                pltpu.sync_copy(den_in.at[pl.ds(8 * _NP + row0 * 8, 128)],
                                dv1)
                for k in range(8):
                    d = dv0[pl.ds(k * 16, 16)] + dv1[pl.ds(k * 16, 16)]
                    inv_c[pl.ds(k * 16, 16)] = 1.0 / (d + 1e-16)

                def _row(r, _):
                    iv = inv_c[pl.ds((r >> 1) * 16, 16)]
                    base = (r & 1) * 8
                    if last:
                        for v in range(C // 16):
                            acc = bias_c[pl.ds(v * 16, 16)]
                            for h in range(HEADS):
                                sp = jnp.take(iv, _spl(0) + (base + h)) * 0.125
                                acc = acc + fin[r, pl.ds(h * C + v * 16, 16)] * sp
                            fin[r, pl.ds(v * 16, 16)] = acc
                    else:
                        for h in range(HEADS):
                            sp = jnp.take(iv, _spl(0) + (base + h))
                            for v in range(C // 16):
                                off = h * C + v * 16
                                fin[r, pl.ds(off, 16)] = (
                                    fin[r, pl.ds(off, 16)] * sp
                                    + bias_c[pl.ds(off, 16)])
                    return 0
                lax.fori_loop(0, 16, _row, 0)
                if last:
                    pltpu.sync_copy(fin.at[:, pl.ds(0, C)],
                                    xout.at[pl.ds(row0, 16)])
                else:
                    pltpu.sync_copy(fin, xout.at[pl.ds(row0, 16)])
            return 0
        lax.fori_loop(0, 1, _fingrp, 0)
        plsc.subcore_barrier()
        return 0

    lax.fori_loop(0, _NB // 2, _block, 0)


def _out_pass(srcp, dstp, exp_e, h, den, offs, bias, last):
    odim = C if last else HC
    f = pl.kernel(
        functools.partial(_out_body, last=last),
        out_type=jax.ShapeDtypeStruct((N, odim), jnp.float32),
        mesh=_sc_mesh,
        compiler_params=_sc_params,
        scratch_types=[
            pltpu.VMEM((_K4,), jnp.int32),
            pltpu.VMEM((_K4,), jnp.int32),
            pltpu.VMEM((_K4,), jnp.int32),
            pltpu.VMEM((_K4 * 8,), jnp.float32),
            pltpu.VMEM((_K4, HC), jnp.float32),
            pltpu.VMEM((_K4, HC), jnp.float32),
            pltpu.VMEM((16, HC), jnp.float32),
            pltpu.VMEM((8, HC), jnp.float32),
            pltpu.VMEM((48,), jnp.int32),
            pltpu.VMEM((HC,), jnp.float32),
            pltpu.VMEM((128,), jnp.float32),
            pltpu.VMEM((128,), jnp.float32),
            pltpu.VMEM((128,), jnp.float32),
            pltpu.SemaphoreType.DMA,
            pltpu.SemaphoreType.DMA,
            pltpu.VMEM_SHARED((_BS + 1, HC), jnp.float32),
        ],
    )
    return f(srcp, dstp, exp_e, h, den, offs, bias)


def _bucket_edges(src, dst, edge_attr):
    blk = dst >> 8
    counts = jnp.bincount(blk, length=_NB)
    caps = ((counts + 1023) // 1024) * 1024
    offs = jnp.concatenate([jnp.zeros((1,), counts.dtype),
                            jnp.cumsum(caps)]).astype(jnp.int32)
    order = jnp.argsort(blk, stable=True)
    sblk = blk[order]
    cstart = jnp.concatenate([jnp.zeros((1,), counts.dtype),
                              jnp.cumsum(counts)])[:-1]
    pos = offs[sblk] + (jnp.arange(E, dtype=jnp.int32)
                        - cstart[sblk].astype(jnp.int32))
    srcp = jnp.zeros((_EP,), jnp.int32).at[pos].set(src[order])
    dstp = jnp.full((_EP,), _DUMMY, jnp.int32).at[pos].set(dst[order])
    eap = jnp.zeros((_EP, EDGE_DIM), jnp.float32).at[pos].set(edge_attr[order])
    offs32 = jnp.zeros((48,), jnp.int32).at[:_NB + 1].set(offs)
    return srcp, dstp, eap, offs32


def kernel(x, edge_index, edge_attr,
           W1, We1, as1, ad1, ae1, b1,
           W2, We2, as2, ad2, ae2, b2,
           W3, We3, as3, ad3, ae3, b3):
    src = edge_index[0]
    dst = edge_index[1]
    srcp, dstp, eap, offs32 = _bucket_edges(src, dst, edge_attr)
    srcp2 = srcp.reshape(_EP // 128, 128)
    dstp2 = dstp.reshape(_EP // 128, 128)

    Wc1, Ve1 = _fold_weights(W1, We1, as1, ad1, ae1)
    Wc2, Ve2 = _fold_weights(W2, We2, as2, ad2, ae2)
    Wc3, Ve3 = _fold_weights(W3, We3, as3, ad3, ae3)
    Vcat = jnp.concatenate([Ve1, Ve2, Ve3], axis=1)

    ae_all = _edge_logits(eap, Vcat)

    xin = x
    for li, (Wc, bias) in enumerate(((Wc1, b1), (Wc2, b2), (Wc3, b3))):
        h_all = _node_matmul(xin, Wc, relu=(li > 0))
        h = h_all[:, :HC]
        asrc = h_all[:, HC:HC + 16]
        adst = jnp.pad(h_all[:, HC + 16:HC + 32], ((0, _NP - N), (0, 0)))
        exp_e, den = _alpha_pass(srcp2, dstp2, ae_all[li], asrc, adst)
        bias_p = bias if li < 2 else jnp.pad(bias, (0, HC - C))
        xin = _out_pass(srcp, dstp, exp_e, h, den, offs32, bias_p,
                        last=(li == 2))
    return xin


# speedup vs baseline: 5.9229x; 1.0176x over previous
import functools

import jax
import jax.numpy as jnp
from jax import lax
from jax.experimental import pallas as pl
from jax.experimental.pallas import tpu as pltpu
from jax.experimental.pallas import tpu_sc as plsc

N = 10000
E = 320000
HEADS = 8
C = 128
HC = HEADS * C
EDGE_DIM = 32

_M_TILE = 400
_E_TILE = 2048

_NC = 2
_NS = 16
_NW = _NC * _NS

_BS = 256
_NB = 40
_EP = 393216
_EPT = _EP // _NW
_K3 = 1024
_NCH3 = _EPT // _K3
_K4 = 32
_NP = 10240
_DUMMY = N

_sc_mesh = plsc.VectorSubcoreMesh(
    core_axis_name="c", subcore_axis_name="s", num_cores=_NC, num_subcores=_NS)
_sc_params = pltpu.CompilerParams(use_tc_tiling_on_sc=False)


def _i16():
    return lax.broadcasted_iota(jnp.int32, (16,), 0)


def _spl(v):
    return jnp.full((16,), v, jnp.int32)


def _comb8(a, b):
    i = _i16()
    return jnp.where(i < 8, a, jnp.take(b, i & 7))


def _fold_weights(W, We, a_s, a_d, a_e):
    d = W.shape[0]
    Was = (W.reshape(d, HEADS, C) * a_s[None]).sum(-1)
    Wad = (W.reshape(d, HEADS, C) * a_d[None]).sum(-1)
    Ve = (We.reshape(EDGE_DIM, HEADS, C) * a_e[None]).sum(-1)
    z8 = jnp.zeros((d, 8), jnp.float32)
    pad = jnp.zeros((d, 1152 - HC - 32), jnp.float32)
    Wcat = jnp.concatenate([W, Was, z8, Wad, z8, pad], axis=1)
    Ve16 = jnp.concatenate([Ve, jnp.zeros((EDGE_DIM, 8), jnp.float32)], axis=1)
    return Wcat, Ve16


def _mm_body(x_ref, w_ref, h_ref, l_ref, *, relu):
    xb = x_ref[...]
    if relu:
        xb = jnp.maximum(xb, 0.0)
    d = jnp.dot(xb, w_ref[...], preferred_element_type=jnp.float32)
    h_ref[...] = d[:, :HC]
    l_ref[...] = d[:, HC:]


def _node_matmul(xin, Wcat, relu):
    K = xin.shape[1]
    return pl.pallas_call(
        functools.partial(_mm_body, relu=relu),
        grid=(N // _M_TILE,),
        in_specs=[
            pl.BlockSpec((_M_TILE, K), lambda i: (i, 0)),
            pl.BlockSpec((K, 1152), lambda i: (0, 0)),
        ],
        out_specs=(pl.BlockSpec((_M_TILE, HC), lambda i: (i, 0)),
                   pl.BlockSpec((_M_TILE, 128), lambda i: (i, 0))),
        out_shape=(jax.ShapeDtypeStruct((N, HC), jnp.float32),
                   jax.ShapeDtypeStruct((N, 128), jnp.float32)),
    )(xin, Wcat)


def _ae_body(ea_ref, v_ref, o1_ref, o2_ref, o3_ref):
    d = jnp.dot(ea_ref[...], v_ref[...], preferred_element_type=jnp.float32)
    o1_ref[...] = d[:, 0:16]
    o2_ref[...] = d[:, 16:32]
    o3_ref[...] = d[:, 32:48]


def _edge_logits(eap, Vcat):
    spec = pl.BlockSpec((_E_TILE, 16), lambda i: (i, 0))
    shp = jax.ShapeDtypeStruct((_EP, 16), jnp.float32)
    return pl.pallas_call(
        _ae_body,
        grid=(_EP // _E_TILE,),
        in_specs=[
            pl.BlockSpec((_E_TILE, EDGE_DIM), lambda i: (i, 0)),
            pl.BlockSpec((EDGE_DIM, 48), lambda i: (0, 0)),
        ],
        out_specs=(spec, spec, spec),
        out_shape=(shp, shp, shp),
    )(eap, Vcat)


def _alpha_body(srcp, dstp, aet, asrc, adst,
                exp_out, den_out,
                src_c, dst_c, ae_c, asg, adg, ex_s, didx_c, zrow, den_sp):
    c = lax.axis_index("c")
    s = lax.axis_index("s")
    zero16 = jnp.zeros((16,), jnp.float32)

    def _z(i, _):
        zrow[pl.ds(i * 16, 16)] = zero16
        return 0
    lax.fori_loop(0, 320, _z, 0)
    pltpu.sync_copy(zrow, den_sp.at[pl.ds(s * 5120, 5120)])
    plsc.subcore_barrier()

    gbase = c * (_EP // _NC) + s * _EPT
    i16 = _i16()
    lane7 = i16 & 7

    def _chunk(i, _):
        b = gbase + i * _K3
        br = b // 128
        pltpu.sync_copy(srcp.at[pl.ds(br, 8)], src_c)
        pltpu.sync_copy(dstp.at[pl.ds(br, 8)], dst_c)
        pltpu.sync_copy(aet.at[pl.ds(b, _K3)], ae_c)
        for t in range(8):
            pltpu.sync_copy(asrc.at[src_c.at[t]], asg.at[pl.ds(t * 128, 128)])
        for t in range(8):
            pltpu.sync_copy(adst.at[dst_c.at[t]], adg.at[pl.ds(t * 128, 128)])

        def _pair(m, _):
            av = _comb8(asg[2 * m], asg[2 * m + 1])
            bv = _comb8(adg[2 * m], adg[2 * m + 1])
            ev = _comb8(ae_c[2 * m], ae_c[2 * m + 1])
            al = av + bv + ev
            al = jnp.where(al > 0, al, al * 0.2)
            exv = jnp.exp(al)
            ex_s[pl.ds(m * 16, 16)] = exv
            dstv = dst_c[m >> 6, pl.ds(((m >> 3) & 7) * 16, 16)]
            e0 = (m & 7) * 2
            d0 = jnp.take(dstv, _spl(0) + e0)
            d1 = jnp.take(dstv, _spl(0) + (e0 + 1))
            didx = jnp.where(i16 < 8, d0 * 8 + i16, d1 * 8 + lane7)
            didx_c[m >> 3, pl.ds((m & 7) * 16, 16)] = didx
            return 0
        lax.fori_loop(0, _K3 // 2, _pair, 0)

        def _scat(t, _):
            pltpu.sync_copy(ex_s.at[pl.ds(t * 128, 128)],
                            den_sp.at[didx_c.at[t]], add=True)
            return 0
        lax.fori_loop(0, _K3 * 8 // 128, _scat, 0)
        pltpu.sync_copy(ex_s, exp_out.at[pl.ds(b * 8, _K3 * 8)])
        return 0

    lax.fori_loop(0, _NCH3, _chunk, 0)
    plsc.subcore_barrier()

    @pl.when(s == 0)
    def _():
        pltpu.sync_copy(den_sp, den_out.at[pl.ds(c * 8 * _NP, 8 * _NP)])


def _alpha_pass(srcp, dstp, aet, asrc, adst):
    f = pl.kernel(
        _alpha_body,
        out_type=(jax.ShapeDtypeStruct((_EP * 8,), jnp.float32),
                  jax.ShapeDtypeStruct((2 * 8 * _NP,), jnp.float32)),
        mesh=_sc_mesh,
        compiler_params=_sc_params,
        scratch_types=[
            pltpu.VMEM((8, 128), jnp.int32),
            pltpu.VMEM((8, 128), jnp.int32),
            pltpu.VMEM((_K3, 16), jnp.float32),
            pltpu.VMEM((_K3, 16), jnp.float32),
            pltpu.VMEM((_K3, 16), jnp.float32),
            pltpu.VMEM((_K3 * 8,), jnp.float32),
            pltpu.VMEM((_K3 * 8 // 128, 128), jnp.int32),
            pltpu.VMEM((5120,), jnp.float32),
            pltpu.VMEM_SHARED((8 * _NP,), jnp.float32),
        ],
    )
    return f(srcp, dstp, aet, asrc, adst)


def _out_body(srcp, dstp, exp_in, h_in, den_in, offs, bias,
              xout,
              src_c0, src_c1, drel_c, ex_c, rows0, rows1, fin, zbuf, bvm,
              bias_c, dv0, dv1, inv_c, sem0, sem1, acc_sp, *, last):
    c = lax.axis_index("c")
    s = lax.axis_index("s")
    zero16 = jnp.zeros((16,), jnp.float32)

    def _z(i, _):
        zbuf[i >> 6, pl.ds((i & 63) * 16, 16)] = zero16
        return 0
    lax.fori_loop(0, 512, _z, 0)
    pltpu.sync_copy(offs, bvm)
    pltpu.sync_copy(bias, bias_c)
    bv0 = bvm[pl.ds(0, 16)]
    bv1 = bvm[pl.ds(16, 16)]
    bv2 = bvm[pl.ds(32, 16)]

    _offvals = ([bv0[k] for k in range(16)]
                + [bv1[k] for k in range(16)]
                + [bv2[k] for k in range(_NB + 1 - 32)])

    def _get_off(b):
        v = _offvals[0]
        for k in range(1, _NB + 1):
            v = jnp.where(b == k, _offvals[k], v)
        return v

    def _block(bb, _):
        b = bb * 2 + c
        lo = pl.multiple_of(_get_off(b), 1024)
        hi = pl.multiple_of(_get_off(b + 1), 1024)

        pltpu.sync_copy(zbuf, acc_sp.at[pl.ds(s * 16, 8)])
        pltpu.sync_copy(zbuf, acc_sp.at[pl.ds(s * 16 + 8, 8)])

        @pl.when(s == 0)
        def _():
            pltpu.sync_copy(zbuf.at[pl.ds(0, 1)], acc_sp.at[pl.ds(_BS, 1)])
        plsc.subcore_barrier()

        def _ch_off(k):
            ch = lo + (k * 16 + s) * _K4
            return pl.multiple_of(jnp.minimum(ch, _EP - _K4), _K4)

        def _start(k, src_b, rows_b, sem):
            ch = _ch_off(k)
            pltpu.sync_copy(srcp.at[pl.ds(ch, _K4)], src_b)
            pltpu.async_copy(h_in.at[src_b], rows_b, sem)

        def _do(k, src_b, rows_b, sem):
            pltpu.make_async_copy(h_in.at[src_b], rows_b, sem).wait()
            ch = _ch_off(k)
            pltpu.sync_copy(exp_in.at[pl.ds(ch * 8, _K4 * 8)], ex_c)
            pltpu.sync_copy(dstp.at[pl.ds(ch, _K4)], drel_c)
            for g in range(_K4 // 16):
                dstv = drel_c[pl.ds(g * 16, 16)]
                rel = dstv - b * _BS
                inb = (rel >= 0) & (rel < _BS)
                drel_c[pl.ds(g * 16, 16)] = jnp.where(inb, rel, _BS)

            def _scale(e, _):
                exv = ex_c[pl.ds((e >> 1) * 16, 16)]
                base = (e & 1) * 8
                for h in range(HEADS):
                    sp = jnp.take(exv, _spl(0) + (base + h))
                    for v in range(C // 16):
                        off = h * C + v * 16
                        rows_b[e, pl.ds(off, 16)] = (
                            rows_b[e, pl.ds(off, 16)] * sp)
                return 0
            lax.fori_loop(0, _K4, _scale, 0)
            pltpu.sync_copy(rows_b, acc_sp.at[drel_c], add=True)

        _start(0, src_c0, rows0, sem0)
        _start(1, src_c1, rows1, sem1)

        def _piter(i2, _):
            k0 = i2 * 2
            _do(k0, src_c0, rows0, sem0)
            _start(k0 + 2, src_c0, rows0, sem0)
            _do(k0 + 1, src_c1, rows1, sem1)
            _start(k0 + 3, src_c1, rows1, sem1)
            return 0
        lax.fori_loop(0, (hi - lo) // (32 * _K4), _piter, 0)
        pltpu.make_async_copy(h_in.at[src_c0], rows0, sem0).wait()
        pltpu.make_async_copy(h_in.at[src_c1], rows1, sem1).wait()
        plsc.subcore_barrier()

        rcount = jnp.minimum(_BS, N - b * _BS)

        def _fingrp(q, _):
            r0 = s * 16 + q * 0

            @pl.when(r0 < rcount)
            def _():
                row0 = b * _BS + r0
                pltpu.sync_copy(acc_sp.at[pl.ds(r0, 16)], fin)
                pltpu.sync_copy(den_in.at[pl.ds(row0 * 8, 128)], dv0)
                pltpu.sync_copy(den_in.at[pl.ds(8 * _NP + row0 * 8, 128)],
                                dv1)
                for k in range(8):
                    d = dv0[pl.ds(k * 16, 16)] + dv1[pl.ds(k * 16, 16)]
                    inv_c[pl.ds(k * 16, 16)] = 1.0 / (d + 1e-16)

                def _row(r, _):
                    iv = inv_c[pl.ds((r >> 1) * 16, 16)]
                    base = (r & 1) * 8
                    if last:
                        for v in range(C // 16):
                            acc = bias_c[pl.ds(v * 16, 16)]
                            for h in range(HEADS):
                                sp = jnp.take(iv, _spl(0) + (base + h)) * 0.125
                                acc = acc + fin[r, pl.ds(h * C + v * 16, 16)] * sp
                            fin[r, pl.ds(v * 16, 16)] = acc
                    else:
                        for h in range(HEADS):
                            sp = jnp.take(iv, _spl(0) + (base + h))
                            for v in range(C // 16):
                                off = h * C + v * 16
                                fin[r, pl.ds(off, 16)] = (
                                    fin[r, pl.ds(off, 16)] * sp
                                    + bias_c[pl.ds(off, 16)])
                    return 0
                lax.fori_loop(0, 16, _row, 0)
                if last:
                    pltpu.sync_copy(fin.at[:, pl.ds(0, C)],
                                    xout.at[pl.ds(row0, 16)])
                else:
                    pltpu.sync_copy(fin, xout.at[pl.ds(row0, 16)])
            return 0
        lax.fori_loop(0, 1, _fingrp, 0)
        plsc.subcore_barrier()
        return 0

    lax.fori_loop(0, _NB // 2, _block, 0)


def _out_pass(srcp, dstp, exp_e, h, den, offs, bias, last):
    odim = C if last else HC
    f = pl.kernel(
        functools.partial(_out_body, last=last),
        out_type=jax.ShapeDtypeStruct((N, odim), jnp.float32),
        mesh=_sc_mesh,
        compiler_params=_sc_params,
        scratch_types=[
            pltpu.VMEM((_K4,), jnp.int32),
            pltpu.VMEM((_K4,), jnp.int32),
            pltpu.VMEM((_K4,), jnp.int32),
            pltpu.VMEM((_K4 * 8,), jnp.float32),
            pltpu.VMEM((_K4, HC), jnp.float32),
            pltpu.VMEM((_K4, HC), jnp.float32),
            pltpu.VMEM((16, HC), jnp.float32),
            pltpu.VMEM((8, HC), jnp.float32),
            pltpu.VMEM((48,), jnp.int32),
            pltpu.VMEM((HC,), jnp.float32),
            pltpu.VMEM((128,), jnp.float32),
            pltpu.VMEM((128,), jnp.float32),
            pltpu.VMEM((128,), jnp.float32),
            pltpu.SemaphoreType.DMA,
            pltpu.SemaphoreType.DMA,
            pltpu.VMEM_SHARED((_BS + 1, HC), jnp.float32),
        ],
    )
    return f(srcp, dstp, exp_e, h, den, offs, bias)


def _bucket_edges(src, dst, edge_attr):
    blk = dst >> 8
    counts = jnp.bincount(blk, length=_NB)
    caps = ((counts + 1023) // 1024) * 1024
    offs = jnp.concatenate([jnp.zeros((1,), counts.dtype),
                            jnp.cumsum(caps)]).astype(jnp.int32)
    order = jnp.argsort(blk, stable=True)
    sblk = blk[order]
    cstart = jnp.concatenate([jnp.zeros((1,), counts.dtype),
                              jnp.cumsum(counts)])[:-1]
    pos = offs[sblk] + (jnp.arange(E, dtype=jnp.int32)
                        - cstart[sblk].astype(jnp.int32))
    slot = jnp.full((_EP,), E, jnp.int32).at[pos].set(order)
    src_x = jnp.concatenate([src, jnp.zeros((1,), jnp.int32)])
    dst_x = jnp.concatenate([dst, jnp.full((1,), _DUMMY, jnp.int32)])
    ea_x = jnp.concatenate([edge_attr,
                            jnp.zeros((1, EDGE_DIM), jnp.float32)])
    srcp = src_x[slot]
    dstp = dst_x[slot]
    eap = ea_x[slot]
    offs32 = jnp.zeros((48,), jnp.int32).at[:_NB + 1].set(offs)
    return srcp, dstp, eap, offs32


def kernel(x, edge_index, edge_attr,
           W1, We1, as1, ad1, ae1, b1,
           W2, We2, as2, ad2, ae2, b2,
           W3, We3, as3, ad3, ae3, b3):
    src = edge_index[0]
    dst = edge_index[1]
    srcp, dstp, eap, offs32 = _bucket_edges(src, dst, edge_attr)
    srcp2 = srcp.reshape(_EP // 128, 128)
    dstp2 = dstp.reshape(_EP // 128, 128)

    Wc1, Ve1 = _fold_weights(W1, We1, as1, ad1, ae1)
    Wc2, Ve2 = _fold_weights(W2, We2, as2, ad2, ae2)
    Wc3, Ve3 = _fold_weights(W3, We3, as3, ad3, ae3)
    Vcat = jnp.concatenate([Ve1, Ve2, Ve3], axis=1)

    ae_all = _edge_logits(eap, Vcat)

    xin = x
    for li, (Wc, bias) in enumerate(((Wc1, b1), (Wc2, b2), (Wc3, b3))):
        h, hl = _node_matmul(xin, Wc, relu=(li > 0))
        asrc = hl[:, :16]
        adst = jnp.pad(hl[:, 16:32], ((0, _NP - N), (0, 0)))
        exp_e, den = _alpha_pass(srcp2, dstp2, ae_all[li], asrc, adst)
        bias_p = bias if li < 2 else jnp.pad(bias, (0, HC - C))
        xin = _out_pass(srcp, dstp, exp_e, h, den, offs32, bias_p,
                        last=(li == 2))
    return xin
